# baseline (speedup 1.0000x reference)
import jax
import jax.numpy as jnp
from jax.experimental import pallas as pl
from jax.experimental.pallas import tpu as pltpu

BN_EPS = 1e-5
INTER = jnp.bfloat16
ACC = jnp.float32


def _zero_halo(pad_ref, hp, wp, c):
    zrow = jnp.zeros((1, wp, c), pad_ref.dtype)
    zcol = jnp.zeros((hp, 1, c), pad_ref.dtype)
    pad_ref[0:1, :, :] = zrow
    pad_ref[hp - 1:hp, :, :] = zrow
    pad_ref[:, 0:1, :] = zcol
    pad_ref[:, wp - 1:wp, :] = zcol


def _im2col(pad_ref, ho, wo, c):
    cols = []
    for dy in range(3):
        for dx in range(3):
            cols.append(pad_ref[dy:dy + ho, dx:dx + wo, :])
    return jnp.concatenate(
        cols, axis=-1).reshape(ho * wo, 9 * c).astype(INTER)


def _pool_cf(x_ref, s_ref, o_ref):
    ipb, cin, h, w = x_ref.shape
    ho, wo = h // 2, w // 2
    for i in range(ipb):
        m = jnp.maximum(
            x_ref[pl.ds(i, 1), :, pl.ds(0, ho, 2), :],
            x_ref[pl.ds(i, 1), :, pl.ds(1, ho, 2), :])[0]
        shifted = jnp.concatenate(
            [m[:, :, 1:], jnp.zeros((cin, ho, 1), jnp.float32)], axis=-1)
        mw = jnp.maximum(m, shifted).astype(INTER)
        p = jnp.dot(mw.reshape(cin * ho, w), s_ref[...],
                    preferred_element_type=ACC)
        o_ref[i] = p.reshape(cin, ho, wo).astype(INTER)


def _conv1(p_ref, w_ref, y_ref, st_ref, pad_ref):
    ipb, ho, wo, cin = p_ref.shape
    cmid = w_ref.shape[1]

    for i in range(ipb):
        _zero_halo(pad_ref, ho + 2, wo + 2, cin)
        pad_ref[1:ho + 1, 1:wo + 1, :] = p_ref[i]
        patches = _im2col(pad_ref, ho, wo, cin)
        y = jnp.dot(patches, w_ref[...], preferred_element_type=ACC)
        s = jnp.sum(y, axis=0, keepdims=True)
        ss = jnp.sum(y * y, axis=0, keepdims=True)
        st_ref[i] = jnp.concatenate([s, ss], axis=0)
        y_ref[i] = y.reshape(ho, wo, cmid).astype(y_ref.dtype)


def _bn_relu_conv2(y1_ref, w_ref, sc_ref, sh_ref, y_ref, st_ref, pad_ref):
    ipb, ho, wo, cmid = y1_ref.shape
    cout = w_ref.shape[1]

    for i in range(ipb):
        h1 = jnp.maximum(y1_ref[i].astype(ACC) * sc_ref[...] + sh_ref[...], 0.0)

        _zero_halo(pad_ref, ho + 2, wo + 2, cmid)
        pad_ref[1:ho + 1, 1:wo + 1, :] = h1
        patches = _im2col(pad_ref, ho, wo, cmid)
        y = jax.lax.dot_general(
            w_ref[...], patches, (((0,), (1,)), ((), ())),
            preferred_element_type=ACC)
        s = jnp.sum(y, axis=1, keepdims=True)
        ss = jnp.sum(y * y, axis=1, keepdims=True)
        st_ref[i] = jnp.concatenate([s, ss], axis=1)
        y_ref[i] = y.astype(y_ref.dtype)


def _bn_relu_out(y_ref, sc_ref, sh_ref, o_ref):
    sc = sc_ref[...][None]
    sh = sh_ref[...][None]
    o_ref[...] = jnp.maximum(
        y_ref[...].astype(ACC) * sc + sh, 0.0).astype(o_ref.dtype)


def _fold_bn(sum_nc, sumsq_nc, gamma, beta, count):
    mean = jnp.sum(sum_nc, axis=0) / count
    var = jnp.sum(sumsq_nc, axis=0) / count - mean * mean
    scale = gamma.reshape(-1) * jax.lax.rsqrt(var + BN_EPS)
    shift = beta.reshape(-1) - mean * scale
    return scale, shift


def _images_per_step(n, bytes_per_image, budget=4 << 20):
    for cand in range(n, 0, -1):
        if n % cand == 0 and cand * bytes_per_image <= budget:
            return cand
    return 1


def kernel(x, w1, g1, b1, w2, g2, b2):
    N, Cin, H, W = x.shape
    Ho, Wo = H // 2, W // 2
    Cmid, Cout = w1.shape[-1], w2.shape[-1]
    count = N * Ho * Wo

    w1m = w1.reshape(9 * Cin, Cmid).astype(INTER)
    w2m = w2.reshape(9 * Cmid, Cout).astype(INTER)

    cparams = pltpu.CompilerParams(dimension_semantics=("arbitrary",))
    ipb = 4
    steps = N // ipb

    sel = (jnp.arange(W)[:, None] == 2 * jnp.arange(Wo)[None, :]).astype(INTER)
    pooled_cf = pl.pallas_call(
        _pool_cf,
        grid=(steps,),
        in_specs=[
            pl.BlockSpec((ipb, Cin, H, W), lambda n: (n, 0, 0, 0)),
            pl.BlockSpec((W, Wo), lambda n: (0, 0)),
        ],
        out_specs=pl.BlockSpec((ipb, Cin, Ho, Wo), lambda n: (n, 0, 0, 0)),
        out_shape=jax.ShapeDtypeStruct((N, Cin, Ho, Wo), INTER),
        compiler_params=cparams,
    )(x, sel)
    pooled = jnp.transpose(pooled_cf, (0, 2, 3, 1))

    y1, st1 = pl.pallas_call(
        _conv1,
        grid=(steps,),
        in_specs=[
            pl.BlockSpec((ipb, Ho, Wo, Cin), lambda n: (n, 0, 0, 0)),
            pl.BlockSpec((9 * Cin, Cmid), lambda n: (0, 0)),
        ],
        out_specs=[
            pl.BlockSpec((ipb, Ho, Wo, Cmid), lambda n: (n, 0, 0, 0)),
            pl.BlockSpec((ipb, 2, Cmid), lambda n: (n, 0, 0)),
        ],
        out_shape=[
            jax.ShapeDtypeStruct((N, Ho, Wo, Cmid), INTER),
            jax.ShapeDtypeStruct((N, 2, Cmid), ACC),
        ],
        scratch_shapes=[pltpu.VMEM((Ho + 2, Wo + 2, Cin), INTER)],
        compiler_params=cparams,
    )(pooled, w1m)

    sc1, sh1 = _fold_bn(st1[:, 0, :], st1[:, 1, :], g1, b1, count)

    y2, st2 = pl.pallas_call(
        _bn_relu_conv2,
        grid=(steps,),
        in_specs=[
            pl.BlockSpec((ipb, Ho, Wo, Cmid), lambda n: (n, 0, 0, 0)),
            pl.BlockSpec((9 * Cmid, Cout), lambda n: (0, 0)),
            pl.BlockSpec((1, Cmid), lambda n: (0, 0)),
            pl.BlockSpec((1, Cmid), lambda n: (0, 0)),
        ],
        out_specs=[
            pl.BlockSpec((ipb, Cout, Ho * Wo), lambda n: (n, 0, 0)),
            pl.BlockSpec((ipb, Cout, 2), lambda n: (n, 0, 0)),
        ],
        out_shape=[
            jax.ShapeDtypeStruct((N, Cout, Ho * Wo), INTER),
            jax.ShapeDtypeStruct((N, Cout, 2), ACC),
        ],
        scratch_shapes=[pltpu.VMEM((Ho + 2, Wo + 2, Cmid), jnp.float32)],
        compiler_params=cparams,
    )(y1, w2m, sc1.reshape(1, Cmid), sh1.reshape(1, Cmid))

    sc2, sh2 = _fold_bn(st2[:, :, 0], st2[:, :, 1], g2, b2, count)

    opb = _images_per_step(N, Cout * Ho * Wo * 6, 12 << 20)
    outf = pl.pallas_call(
        _bn_relu_out,
        grid=(N // opb,),
        in_specs=[
            pl.BlockSpec((opb, Cout, Ho * Wo), lambda i: (i, 0, 0)),
            pl.BlockSpec((Cout, 1), lambda i: (0, 0)),
            pl.BlockSpec((Cout, 1), lambda i: (0, 0)),
        ],
        out_specs=pl.BlockSpec((opb, Cout, Ho * Wo), lambda i: (i, 0, 0)),
        out_shape=jax.ShapeDtypeStruct((N, Cout, Ho * Wo), x.dtype),
        compiler_params=cparams,
    )(y2, sc2.reshape(Cout, 1), sh2.reshape(Cout, 1))

    return outf.reshape(N, Cout, Ho, Wo)


# speedup vs baseline: 1.0318x; 1.0318x over previous
import jax
import jax.numpy as jnp
from jax.experimental import pallas as pl
from jax.experimental.pallas import tpu as pltpu

BN_EPS = 1e-5
INTER = jnp.bfloat16
ACC = jnp.float32


def _zero_halo(pad_ref, hp, wp, c):
    zrow = jnp.zeros((1, wp, c), pad_ref.dtype)
    zcol = jnp.zeros((hp, 1, c), pad_ref.dtype)
    pad_ref[0:1, :, :] = zrow
    pad_ref[hp - 1:hp, :, :] = zrow
    pad_ref[:, 0:1, :] = zcol
    pad_ref[:, wp - 1:wp, :] = zcol


def _im2col(pad_ref, ho, wo, c):
    cols = []
    for dy in range(3):
        for dx in range(3):
            cols.append(pad_ref[dy:dy + ho, dx:dx + wo, :])
    return jnp.concatenate(cols, axis=-1).reshape(ho * wo, 9 * c)


def _pool_cf(x_ref, s_ref, o_ref):
    ipb, cin, h, w = x_ref.shape
    ho, wo = h // 2, w // 2
    for i in range(ipb):
        m = jnp.maximum(
            x_ref[pl.ds(i, 1), :, pl.ds(0, ho, 2), :],
            x_ref[pl.ds(i, 1), :, pl.ds(1, ho, 2), :])[0]
        shifted = jnp.concatenate(
            [m[:, :, 1:], jnp.zeros((cin, ho, 1), jnp.float32)], axis=-1)
        mw = jnp.maximum(m, shifted).astype(INTER)
        p = jnp.dot(mw.reshape(cin * ho, w), s_ref[...],
                    preferred_element_type=ACC)
        o_ref[i] = p.reshape(cin, ho, wo).astype(INTER)


def _conv1(p_ref, w_ref, y_ref, st_ref, pad_ref):
    ipb, ho, wo, cin = p_ref.shape
    cmid = w_ref.shape[1]

    for i in range(ipb):
        _zero_halo(pad_ref, ho + 2, wo + 2, cin)
        pad_ref[1:ho + 1, 1:wo + 1, :] = p_ref[i]
        patches = _im2col(pad_ref, ho, wo, cin)
        y = jnp.dot(patches, w_ref[...], preferred_element_type=ACC)
        s = jnp.sum(y, axis=0, keepdims=True)
        ss = jnp.sum(y * y, axis=0, keepdims=True)
        st_ref[i] = jnp.concatenate([s, ss], axis=0)
        y_ref[i] = y.reshape(ho, wo, cmid).astype(y_ref.dtype)


def _bn_relu_conv2(y1_ref, w_ref, sc_ref, sh_ref, y_ref, st_ref, pad_ref):
    ipb, ho, wo, cmid = y1_ref.shape
    cout = w_ref.shape[1]

    for i in range(ipb):
        h1 = jnp.maximum(y1_ref[i].astype(ACC) * sc_ref[...] + sh_ref[...], 0.0)

        _zero_halo(pad_ref, ho + 2, wo + 2, cmid)
        pad_ref[1:ho + 1, 1:wo + 1, :] = h1.astype(INTER)
        patches = _im2col(pad_ref, ho, wo, cmid)
        y = jax.lax.dot_general(
            w_ref[...], patches, (((0,), (1,)), ((), ())),
            preferred_element_type=ACC)
        s = jnp.sum(y, axis=1, keepdims=True)
        ss = jnp.sum(y * y, axis=1, keepdims=True)
        st_ref[i] = jnp.concatenate([s, ss], axis=1)
        y_ref[i] = y.astype(y_ref.dtype)


def _bn_relu_out(y_ref, sc_ref, sh_ref, o_ref):
    sc = sc_ref[...][None]
    sh = sh_ref[...][None]
    o_ref[...] = jnp.maximum(
        y_ref[...].astype(ACC) * sc + sh, 0.0).astype(o_ref.dtype)


def _fold_bn(sum_nc, sumsq_nc, gamma, beta, count):
    mean = jnp.sum(sum_nc, axis=0) / count
    var = jnp.sum(sumsq_nc, axis=0) / count - mean * mean
    scale = gamma.reshape(-1) * jax.lax.rsqrt(var + BN_EPS)
    shift = beta.reshape(-1) - mean * scale
    return scale, shift


def _images_per_step(n, bytes_per_image, budget=4 << 20):
    for cand in range(n, 0, -1):
        if n % cand == 0 and cand * bytes_per_image <= budget:
            return cand
    return 1


def kernel(x, w1, g1, b1, w2, g2, b2):
    N, Cin, H, W = x.shape
    Ho, Wo = H // 2, W // 2
    Cmid, Cout = w1.shape[-1], w2.shape[-1]
    count = N * Ho * Wo

    w1m = w1.reshape(9 * Cin, Cmid).astype(INTER)
    w2m = w2.reshape(9 * Cmid, Cout).astype(INTER)

    cparams = pltpu.CompilerParams(dimension_semantics=("arbitrary",),
                               vmem_limit_bytes=50 * 1024 * 1024)
    ipb = 4
    steps = N // ipb

    sel = (jnp.arange(W)[:, None] == 2 * jnp.arange(Wo)[None, :]).astype(INTER)
    pooled_cf = pl.pallas_call(
        _pool_cf,
        grid=(steps,),
        in_specs=[
            pl.BlockSpec((ipb, Cin, H, W), lambda n: (n, 0, 0, 0)),
            pl.BlockSpec((W, Wo), lambda n: (0, 0)),
        ],
        out_specs=pl.BlockSpec((ipb, Cin, Ho, Wo), lambda n: (n, 0, 0, 0)),
        out_shape=jax.ShapeDtypeStruct((N, Cin, Ho, Wo), INTER),
        compiler_params=cparams,
    )(x, sel)
    pooled = jnp.transpose(pooled_cf, (0, 2, 3, 1))

    y1, st1 = pl.pallas_call(
        _conv1,
        grid=(steps,),
        in_specs=[
            pl.BlockSpec((ipb, Ho, Wo, Cin), lambda n: (n, 0, 0, 0)),
            pl.BlockSpec((9 * Cin, Cmid), lambda n: (0, 0)),
        ],
        out_specs=[
            pl.BlockSpec((ipb, Ho, Wo, Cmid), lambda n: (n, 0, 0, 0)),
            pl.BlockSpec((ipb, 2, Cmid), lambda n: (n, 0, 0)),
        ],
        out_shape=[
            jax.ShapeDtypeStruct((N, Ho, Wo, Cmid), INTER),
            jax.ShapeDtypeStruct((N, 2, Cmid), ACC),
        ],
        scratch_shapes=[pltpu.VMEM((Ho + 2, Wo + 2, Cin), INTER)],
        compiler_params=cparams,
    )(pooled, w1m)

    sc1, sh1 = _fold_bn(st1[:, 0, :], st1[:, 1, :], g1, b1, count)

    y2, st2 = pl.pallas_call(
        _bn_relu_conv2,
        grid=(steps,),
        in_specs=[
            pl.BlockSpec((ipb, Ho, Wo, Cmid), lambda n: (n, 0, 0, 0)),
            pl.BlockSpec((9 * Cmid, Cout), lambda n: (0, 0)),
            pl.BlockSpec((1, Cmid), lambda n: (0, 0)),
            pl.BlockSpec((1, Cmid), lambda n: (0, 0)),
        ],
        out_specs=[
            pl.BlockSpec((ipb, Cout, Ho * Wo), lambda n: (n, 0, 0)),
            pl.BlockSpec((ipb, Cout, 2), lambda n: (n, 0, 0)),
        ],
        out_shape=[
            jax.ShapeDtypeStruct((N, Cout, Ho * Wo), INTER),
            jax.ShapeDtypeStruct((N, Cout, 2), ACC),
        ],
        scratch_shapes=[pltpu.VMEM((Ho + 2, Wo + 2, Cmid), INTER)],
        compiler_params=cparams,
    )(y1, w2m, sc1.reshape(1, Cmid), sh1.reshape(1, Cmid))

    sc2, sh2 = _fold_bn(st2[:, :, 0], st2[:, :, 1], g2, b2, count)

    opb = _images_per_step(N, Cout * Ho * Wo * 6, 12 << 20)
    outf = pl.pallas_call(
        _bn_relu_out,
        grid=(N // opb,),
        in_specs=[
            pl.BlockSpec((opb, Cout, Ho * Wo), lambda i: (i, 0, 0)),
            pl.BlockSpec((Cout, 1), lambda i: (0, 0)),
            pl.BlockSpec((Cout, 1), lambda i: (0, 0)),
        ],
        out_specs=pl.BlockSpec((opb, Cout, Ho * Wo), lambda i: (i, 0, 0)),
        out_shape=jax.ShapeDtypeStruct((N, Cout, Ho * Wo), x.dtype),
        compiler_params=cparams,
    )(y2, sc2.reshape(Cout, 1), sh2.reshape(Cout, 1))

    return outf.reshape(N, Cout, Ho, Wo)


# speedup vs baseline: 1.0349x; 1.0030x over previous
import jax
import jax.numpy as jnp
from jax.experimental import pallas as pl
from jax.experimental.pallas import tpu as pltpu

BN_EPS = 1e-5
INTER = jnp.bfloat16
ACC = jnp.float32


def _zero_halo(pad_ref, hp, wp, c):
    zrow = jnp.zeros((1, wp, c), pad_ref.dtype)
    zcol = jnp.zeros((hp, 1, c), pad_ref.dtype)
    pad_ref[0:1, :, :] = zrow
    pad_ref[hp - 1:hp, :, :] = zrow
    pad_ref[:, 0:1, :] = zcol
    pad_ref[:, wp - 1:wp, :] = zcol


def _im2col(pad_ref, ho, wo, c):
    cols = []
    for dy in range(3):
        for dx in range(3):
            cols.append(pad_ref[dy:dy + ho, dx:dx + wo, :])
    return jnp.concatenate(cols, axis=-1).reshape(ho * wo, 9 * c)


def _pool_cf(x_ref, s_ref, o_ref):
    ipb, cin, h, w = x_ref.shape
    ho, wo = h // 2, w // 2
    for i in range(ipb):
        m = jnp.maximum(
            x_ref[pl.ds(i, 1), :, pl.ds(0, ho, 2), :],
            x_ref[pl.ds(i, 1), :, pl.ds(1, ho, 2), :])[0]
        shifted = jnp.concatenate(
            [m[:, :, 1:], jnp.zeros((cin, ho, 1), jnp.float32)], axis=-1)
        mw = jnp.maximum(m, shifted).astype(INTER)
        p = jnp.dot(mw.reshape(cin * ho, w), s_ref[...],
                    preferred_element_type=ACC)
        o_ref[i] = p.reshape(cin, ho, wo).astype(INTER)


def _conv1(p_ref, w_ref, y_ref, st_ref, pad_ref):
    ipb, ho, wo, cin = p_ref.shape
    cmid = w_ref.shape[1]

    for i in range(ipb):
        _zero_halo(pad_ref, ho + 2, wo + 2, cin)
        pad_ref[1:ho + 1, 1:wo + 1, :] = p_ref[i]
        patches = _im2col(pad_ref, ho, wo, cin)
        y = jnp.dot(patches, w_ref[...], preferred_element_type=ACC)
        s = jnp.sum(y, axis=0, keepdims=True)
        ss = jnp.sum(y * y, axis=0, keepdims=True)
        st_ref[i] = jnp.concatenate([s, ss], axis=0)
        y_ref[i] = y.reshape(ho, wo, cmid).astype(y_ref.dtype)


def _bn_relu_conv2(y1_ref, w_ref, sc_ref, sh_ref, y_ref, st_ref, pad_ref):
    ipb, ho, wo, cmid = y1_ref.shape
    cout = w_ref.shape[1]

    for i in range(ipb):
        h1 = jnp.maximum(y1_ref[i].astype(ACC) * sc_ref[...] + sh_ref[...], 0.0)

        _zero_halo(pad_ref, ho + 2, wo + 2, cmid)
        pad_ref[1:ho + 1, 1:wo + 1, :] = h1.astype(INTER)
        patches = _im2col(pad_ref, ho, wo, cmid)
        y = jax.lax.dot_general(
            w_ref[...], patches, (((0,), (1,)), ((), ())),
            preferred_element_type=ACC)
        s = jnp.sum(y, axis=1, keepdims=True)
        ss = jnp.sum(y * y, axis=1, keepdims=True)
        st_ref[i] = jnp.concatenate([s, ss], axis=1)
        y_ref[i] = y.astype(y_ref.dtype)


def _bn_relu_out(y_ref, sc_ref, sh_ref, o_ref):
    sc = sc_ref[...][None]
    sh = sh_ref[...][None]
    o_ref[...] = jnp.maximum(
        y_ref[...].astype(ACC) * sc + sh, 0.0).astype(o_ref.dtype)


def _fold_bn(sum_nc, sumsq_nc, gamma, beta, count):
    mean = jnp.sum(sum_nc, axis=0) / count
    var = jnp.sum(sumsq_nc, axis=0) / count - mean * mean
    scale = gamma.reshape(-1) * jax.lax.rsqrt(var + BN_EPS)
    shift = beta.reshape(-1) - mean * scale
    return scale, shift


def _images_per_step(n, bytes_per_image, budget=4 << 20):
    for cand in range(n, 0, -1):
        if n % cand == 0 and cand * bytes_per_image <= budget:
            return cand
    return 1


def kernel(x, w1, g1, b1, w2, g2, b2):
    N, Cin, H, W = x.shape
    Ho, Wo = H // 2, W // 2
    Cmid, Cout = w1.shape[-1], w2.shape[-1]
    count = N * Ho * Wo

    w1m = w1.reshape(9 * Cin, Cmid).astype(INTER)
    w2m = w2.reshape(9 * Cmid, Cout).astype(INTER)

    cparams = pltpu.CompilerParams(dimension_semantics=("arbitrary",),
                               vmem_limit_bytes=50 * 1024 * 1024)
    ipb = 4
    steps = N // ipb

    sel = (jnp.arange(W)[:, None] == 2 * jnp.arange(Wo)[None, :]).astype(INTER)
    pooled_cf = pl.pallas_call(
        _pool_cf,
        grid=(steps,),
        in_specs=[
            pl.BlockSpec((ipb, Cin, H, W), lambda n: (n, 0, 0, 0)),
            pl.BlockSpec((W, Wo), lambda n: (0, 0)),
        ],
        out_specs=pl.BlockSpec((ipb, Cin, Ho, Wo), lambda n: (n, 0, 0, 0)),
        out_shape=jax.ShapeDtypeStruct((N, Cin, Ho, Wo), INTER),
        compiler_params=cparams,
    )(x, sel)
    pooled = jnp.transpose(pooled_cf, (0, 2, 3, 1))

    y1, st1 = pl.pallas_call(
        _conv1,
        grid=(steps,),
        in_specs=[
            pl.BlockSpec((ipb, Ho, Wo, Cin), lambda n: (n, 0, 0, 0)),
            pl.BlockSpec((9 * Cin, Cmid), lambda n: (0, 0)),
        ],
        compiler_params=pltpu.CompilerParams(
            dimension_semantics=("arbitrary",),
            vmem_limit_bytes=50 * 1024 * 1024,
            allow_input_fusion=[True, False]),
        out_specs=[
            pl.BlockSpec((ipb, Ho, Wo, Cmid), lambda n: (n, 0, 0, 0)),
            pl.BlockSpec((ipb, 2, Cmid), lambda n: (n, 0, 0)),
        ],
        out_shape=[
            jax.ShapeDtypeStruct((N, Ho, Wo, Cmid), INTER),
            jax.ShapeDtypeStruct((N, 2, Cmid), ACC),
        ],
        scratch_shapes=[pltpu.VMEM((Ho + 2, Wo + 2, Cin), INTER)],
    )(pooled, w1m)

    sc1, sh1 = _fold_bn(st1[:, 0, :], st1[:, 1, :], g1, b1, count)

    y2, st2 = pl.pallas_call(
        _bn_relu_conv2,
        grid=(steps,),
        in_specs=[
            pl.BlockSpec((ipb, Ho, Wo, Cmid), lambda n: (n, 0, 0, 0)),
            pl.BlockSpec((9 * Cmid, Cout), lambda n: (0, 0)),
            pl.BlockSpec((1, Cmid), lambda n: (0, 0)),
            pl.BlockSpec((1, Cmid), lambda n: (0, 0)),
        ],
        out_specs=[
            pl.BlockSpec((ipb, Cout, Ho * Wo), lambda n: (n, 0, 0)),
            pl.BlockSpec((ipb, Cout, 2), lambda n: (n, 0, 0)),
        ],
        out_shape=[
            jax.ShapeDtypeStruct((N, Cout, Ho * Wo), INTER),
            jax.ShapeDtypeStruct((N, Cout, 2), ACC),
        ],
        scratch_shapes=[pltpu.VMEM((Ho + 2, Wo + 2, Cmid), INTER)],
        compiler_params=cparams,
    )(y1, w2m, sc1.reshape(1, Cmid), sh1.reshape(1, Cmid))

    sc2, sh2 = _fold_bn(st2[:, :, 0], st2[:, :, 1], g2, b2, count)

    opb = _images_per_step(N, Cout * Ho * Wo * 6, 12 << 20)
    outf = pl.pallas_call(
        _bn_relu_out,
        grid=(N // opb,),
        in_specs=[
            pl.BlockSpec((opb, Cout, Ho * Wo), lambda i: (i, 0, 0)),
            pl.BlockSpec((Cout, 1), lambda i: (0, 0)),
            pl.BlockSpec((Cout, 1), lambda i: (0, 0)),
        ],
        out_specs=pl.BlockSpec((opb, Cout, Ho * Wo), lambda i: (i, 0, 0)),
        out_shape=jax.ShapeDtypeStruct((N, Cout, Ho * Wo), x.dtype),
        compiler_params=cparams,
    )(y2, sc2.reshape(Cout, 1), sh2.reshape(Cout, 1))

    return outf.reshape(N, Cout, Ho, Wo)
